# baseline (speedup 1.0000x reference)
import functools

import jax
import jax.numpy as jnp
from jax import lax
from jax.experimental import pallas as pl
from jax.experimental.pallas import tpu as pltpu
from jax.experimental.pallas import tpu_sc as plsc

DIM = 33
NLUT = DIM * DIM * DIM
B, H, W = 8, 512, 512
NW = 32
TILES_PER_IMG = NW // B
ROWS_PER_TILE = H // TILES_PER_IMG
RCHUNK = 1
NCHUNK = ROWS_PER_TILE // RCHUNK
HALF = NCHUNK // 2
GROUPS = W // 16

NLUT_PAD = NLUT + 7
_CORNER_OFF = (0, 1, DIM, DIM + 1, DIM * DIM, DIM * DIM + 1,
               DIM * DIM + DIM, DIM * DIM + DIM + 1)


def _body(x_hbm, lut_hbm, out_hbm,
          lut_v, xbufs, ybufs,
          sem_lut, sem_in0, sem_in1, sem_out0, sem_out1):
    sem_in = (sem_in0, sem_in1)
    sem_out = (sem_out0, sem_out1)

    wid = lax.axis_index("s") * 2 + lax.axis_index("c")
    img = wid // TILES_PER_IMG
    quarter = wid % TILES_PER_IMG
    row_base = quarter * ROWS_PER_TILE

    pltpu.make_async_copy(lut_hbm, lut_v, sem_lut).start()

    def in_desc(ci, p):
        r0 = row_base + ci * RCHUNK
        return pltpu.make_async_copy(
            x_hbm.at[img, :, pl.ds(r0, RCHUNK), :], xbufs[p], sem_in[p])

    def out_desc(ci, p):
        r0 = row_base + ci * RCHUNK
        return pltpu.make_async_copy(
            ybufs[p], out_hbm.at[img, :, pl.ds(r0, RCHUNK), :], sem_out[p])

    in_desc(0, 0).start()
    in_desc(1, 1).start()
    pltpu.make_async_copy(lut_hbm, lut_v, sem_lut).wait()

    lut_c0 = lut_v.at[pl.ds(0, NLUT_PAD)]
    lut_c1 = lut_v.at[pl.ds(NLUT_PAD, NLUT_PAD)]
    lut_c2 = lut_v.at[pl.ds(2 * NLUT_PAD, NLUT_PAD)]

    def compute_chunk(p):
        xbuf = xbufs[p]
        ybuf = ybufs[p]

        @plsc.parallel_loop(0, GROUPS, 1, unroll=1)
        def group_body(gi):
            s = gi * 16
            for row in range(RCHUNK):
                r = xbuf[0, row, pl.ds(s, 16)]
                g = xbuf[1, row, pl.ds(s, 16)]
                b = xbuf[2, row, pl.ds(s, 16)]
                rf = r * jnp.float32(DIM - 1)
                gf = g * jnp.float32(DIM - 1)
                bf = b * jnp.float32(DIM - 1)
                ri = rf.astype(jnp.int32)
                gi_ = gf.astype(jnp.int32)
                bi = bf.astype(jnp.int32)
                rd = rf - ri.astype(jnp.float32)
                gd = gf - gi_.astype(jnp.float32)
                bd = bf - bi.astype(jnp.float32)
                base = ri + gi_ * DIM + bi * (DIM * DIM)
                omr = 1.0 - rd
                omg = 1.0 - gd
                omb = 1.0 - bd
                a00 = omr * omg
                a10 = rd * omg
                a01 = omr * gd
                a11 = rd * gd
                ws = (a00 * omb, a10 * omb, a01 * omb, a11 * omb,
                      a00 * bd, a10 * bd, a01 * bd, a11 * bd)
                idxs = [base + o if o else base for o in _CORNER_OFF]
                acc0 = jnp.zeros((16,), jnp.float32)
                acc1 = jnp.zeros((16,), jnp.float32)
                acc2 = jnp.zeros((16,), jnp.float32)
                for k in range(8):
                    acc0 = acc0 + ws[k] * plsc.load_gather(lut_c0, [idxs[k]])
                    acc1 = acc1 + ws[k] * plsc.load_gather(lut_c1, [idxs[k]])
                    acc2 = acc2 + ws[k] * plsc.load_gather(lut_c2, [idxs[k]])
                ybuf[0, row, pl.ds(s, 16)] = acc0
                ybuf[1, row, pl.ds(s, 16)] = acc1
                ybuf[2, row, pl.ds(s, 16)] = acc2

    def half_body(i, p):
        ci = 2 * i + p
        in_desc(ci, p).wait()

        @pl.when(i >= 1)
        def _():
            out_desc(ci - 2, p).wait()

        compute_chunk(p)
        out_desc(ci, p).start()

        @pl.when(i < HALF - 1)
        def _():
            in_desc(ci + 2, p).start()

    def loop_body(i, _):
        half_body(i, 0)
        half_body(i, 1)
        return 0

    lax.fori_loop(0, HALF, loop_body, 0)
    for p in range(2):
        out_desc(NCHUNK - 2 + p, p).wait()


def kernel(x, LUT):
    k = functools.partial(
        pl.kernel,
        out_type=jax.ShapeDtypeStruct((B, 3, H, W), jnp.float32),
        mesh=plsc.VectorSubcoreMesh(core_axis_name="c", subcore_axis_name="s"),
        compiler_params=pltpu.CompilerParams(needs_layout_passes=False),
        scratch_types=[
            pltpu.VMEM((3 * NLUT_PAD,), jnp.float32),
            [pltpu.VMEM((3, RCHUNK, W), jnp.float32)] * 2,
            [pltpu.VMEM((3, RCHUNK, W), jnp.float32)] * 2,
            pltpu.SemaphoreType.DMA,
            pltpu.SemaphoreType.DMA,
            pltpu.SemaphoreType.DMA,
            pltpu.SemaphoreType.DMA,
            pltpu.SemaphoreType.DMA,
        ],
    )(_body)
    lut_pad = jnp.pad(LUT.reshape(3, NLUT), ((0, 0), (0, NLUT_PAD - NLUT)))
    out = k(x, lut_pad.reshape(-1))
    return out


# speedup vs baseline: 1.0355x; 1.0355x over previous
import functools

import jax
import jax.numpy as jnp
from jax import lax
from jax.experimental import pallas as pl
from jax.experimental.pallas import tpu as pltpu
from jax.experimental.pallas import tpu_sc as plsc

DIM = 33
NLUT = DIM * DIM * DIM
B, H, W = 8, 512, 512
NW = 32
TILES_PER_IMG = NW // B
ROWS_PER_TILE = H // TILES_PER_IMG
RCHUNK = 2
NCHUNK = ROWS_PER_TILE // RCHUNK
HALF = NCHUNK // 2
GROUPS = W // 16

NLUT_PAD = NLUT + 7
_CORNER_OFF = (0, 1, DIM, DIM + 1, DIM * DIM, DIM * DIM + 1,
               DIM * DIM + DIM, DIM * DIM + DIM + 1)


def _body(x_hbm, lut_hbm, out_hbm,
          lut_v, xbufs, ybufs,
          sem_lut, sem_in0, sem_in1, sem_out0, sem_out1):
    sem_in = (sem_in0, sem_in1)
    sem_out = (sem_out0, sem_out1)

    wid = lax.axis_index("s") * 2 + lax.axis_index("c")
    img = wid // TILES_PER_IMG
    quarter = wid % TILES_PER_IMG
    row_base = quarter * ROWS_PER_TILE

    pltpu.make_async_copy(lut_hbm, lut_v, sem_lut).start()

    def in_desc(ci, p):
        r0 = row_base + ci * RCHUNK
        return pltpu.make_async_copy(
            x_hbm.at[img, :, pl.ds(r0, RCHUNK), :], xbufs[p], sem_in[p])

    def out_desc(ci, p):
        r0 = row_base + ci * RCHUNK
        return pltpu.make_async_copy(
            ybufs[p], out_hbm.at[img, :, pl.ds(r0, RCHUNK), :], sem_out[p])

    in_desc(0, 0).start()
    in_desc(1, 1).start()
    pltpu.make_async_copy(lut_hbm, lut_v, sem_lut).wait()

    lut_c0 = lut_v.at[pl.ds(0, NLUT_PAD)]
    lut_c1 = lut_v.at[pl.ds(NLUT_PAD, NLUT_PAD)]
    lut_c2 = lut_v.at[pl.ds(2 * NLUT_PAD, NLUT_PAD)]

    def compute_chunk(p):
        xbuf = xbufs[p]
        ybuf = ybufs[p]

        @plsc.parallel_loop(0, GROUPS, 1, unroll=1)
        def group_body(gi):
            s = gi * 16
            for row in range(RCHUNK):
                r = xbuf[0, row, pl.ds(s, 16)]
                g = xbuf[1, row, pl.ds(s, 16)]
                b = xbuf[2, row, pl.ds(s, 16)]
                rf = r * jnp.float32(DIM - 1)
                gf = g * jnp.float32(DIM - 1)
                bf = b * jnp.float32(DIM - 1)
                ri = rf.astype(jnp.int32)
                gi_ = gf.astype(jnp.int32)
                bi = bf.astype(jnp.int32)
                rd = rf - ri.astype(jnp.float32)
                gd = gf - gi_.astype(jnp.float32)
                bd = bf - bi.astype(jnp.float32)
                base = ri + gi_ * DIM + bi * (DIM * DIM)
                omr = 1.0 - rd
                omg = 1.0 - gd
                omb = 1.0 - bd
                a00 = omr * omg
                a10 = rd * omg
                a01 = omr * gd
                a11 = rd * gd
                ws = (a00 * omb, a10 * omb, a01 * omb, a11 * omb,
                      a00 * bd, a10 * bd, a01 * bd, a11 * bd)
                idxs = [base + o if o else base for o in _CORNER_OFF]
                acc0 = jnp.zeros((16,), jnp.float32)
                acc1 = jnp.zeros((16,), jnp.float32)
                acc2 = jnp.zeros((16,), jnp.float32)
                for k in range(8):
                    acc0 = acc0 + ws[k] * plsc.load_gather(lut_c0, [idxs[k]])
                    acc1 = acc1 + ws[k] * plsc.load_gather(lut_c1, [idxs[k]])
                    acc2 = acc2 + ws[k] * plsc.load_gather(lut_c2, [idxs[k]])
                ybuf[0, row, pl.ds(s, 16)] = acc0
                ybuf[1, row, pl.ds(s, 16)] = acc1
                ybuf[2, row, pl.ds(s, 16)] = acc2

    def half_body(i, p):
        ci = 2 * i + p
        in_desc(ci, p).wait()

        @pl.when(i >= 1)
        def _():
            out_desc(ci - 2, p).wait()

        compute_chunk(p)
        out_desc(ci, p).start()

        @pl.when(i < HALF - 1)
        def _():
            in_desc(ci + 2, p).start()

    def loop_body(i, _):
        half_body(i, 0)
        half_body(i, 1)
        return 0

    lax.fori_loop(0, HALF, loop_body, 0)
    for p in range(2):
        out_desc(NCHUNK - 2 + p, p).wait()


def kernel(x, LUT):
    k = functools.partial(
        pl.kernel,
        out_type=jax.ShapeDtypeStruct((B, 3, H, W), jnp.float32),
        mesh=plsc.VectorSubcoreMesh(core_axis_name="c", subcore_axis_name="s"),
        compiler_params=pltpu.CompilerParams(needs_layout_passes=False),
        scratch_types=[
            pltpu.VMEM((3 * NLUT_PAD,), jnp.float32),
            [pltpu.VMEM((3, RCHUNK, W), jnp.float32)] * 2,
            [pltpu.VMEM((3, RCHUNK, W), jnp.float32)] * 2,
            pltpu.SemaphoreType.DMA,
            pltpu.SemaphoreType.DMA,
            pltpu.SemaphoreType.DMA,
            pltpu.SemaphoreType.DMA,
            pltpu.SemaphoreType.DMA,
        ],
    )(_body)
    lut_pad = jnp.pad(LUT.reshape(3, NLUT), ((0, 0), (0, NLUT_PAD - NLUT)))
    out = k(x, lut_pad.reshape(-1))
    return out
